# baseline (speedup 1.0000x reference)
import functools

import jax
import jax.numpy as jnp
from jax import lax
from jax.experimental import pallas as pl
from jax.experimental.pallas import tpu as pltpu
from jax.experimental.pallas import tpu_sc as plsc

VOCAB = 100000
EMBED_DIM = 128
HIDDEN = 128
NUM_CLASSES = 2
BATCH = 4096
SEQ = 200
SEQ_PAD = 208
HALF = SEQ_PAD // 2

NUM_CORES = 2
NUM_SUBCORES = 16
NUM_WORKERS = NUM_CORES * NUM_SUBCORES
ROWS_PER_WORKER = BATCH // NUM_WORKERS

D_CHUNKS = EMBED_DIM // 16


def _sc_pool_body(x_hbm, table_hbm, out_hbm, idx_v, buf_v, acc_v, sem):
    c = lax.axis_index("c")
    s = lax.axis_index("s")
    wid = s * NUM_CORES + c
    base = wid * ROWS_PER_WORKER

    def row_body(r, carry):
        row = base + r
        pltpu.sync_copy(x_hbm.at[row], idx_v)
        cp0 = pltpu.async_copy(
            table_hbm.at[idx_v.at[pl.ds(0, HALF)]],
            buf_v.at[pl.ds(0, HALF)], sem)
        cp1 = pltpu.async_copy(
            table_hbm.at[idx_v.at[pl.ds(HALF, HALF)]],
            buf_v.at[pl.ds(HALF, HALF)], sem)
        cp0.wait()
        cp1.wait()

        def tok_body(t, accs):
            return tuple(
                accs[ci] + buf_v[t, pl.ds(ci * 16, 16)]
                for ci in range(D_CHUNKS))

        zero = jnp.zeros((16,), jnp.float32)
        accs = lax.fori_loop(0, SEQ_PAD, tok_body,
                             tuple(zero for _ in range(D_CHUNKS)))
        for ci in range(D_CHUNKS):
            acc_v[pl.ds(ci * 16, 16)] = accs[ci]
        pltpu.sync_copy(acc_v, out_hbm.at[row])
        return carry

    lax.fori_loop(0, ROWS_PER_WORKER, row_body, 0)


@jax.jit
def _sc_pool(safe_x, table):
    mesh = plsc.VectorSubcoreMesh(core_axis_name="c", subcore_axis_name="s")
    f = functools.partial(
        pl.kernel, mesh=mesh,
        out_type=jax.ShapeDtypeStruct((BATCH, EMBED_DIM), jnp.float32),
        scratch_types=[
            pltpu.VMEM((SEQ_PAD,), jnp.int32),
            pltpu.VMEM((SEQ_PAD, EMBED_DIM), jnp.float32),
            pltpu.VMEM((EMBED_DIM,), jnp.float32),
            pltpu.SemaphoreType.DMA,
        ],
    )(_sc_pool_body)
    return f(safe_x, table)


def _mlp_body(acc_ref, mask_ref, row0_ref, w1t_ref, b1_ref, g_ref, bt_ref,
              w2t_ref, b2_ref, out_ref):
    acc = acc_ref[...]
    maskf = mask_ref[...].astype(jnp.float32)
    valid = SEQ - jnp.sum(maskf, axis=1, keepdims=True)
    sentinel_cnt = SEQ_PAD - valid
    lengths = jnp.maximum(valid, 1.0)
    sv = (acc - sentinel_cnt * row0_ref[...]) / lengths
    h = jnp.dot(sv, w1t_ref[...], preferred_element_type=jnp.float32)
    h = h + b1_ref[...]
    mu = jnp.mean(h, axis=-1, keepdims=True)
    var = jnp.mean(jnp.square(h), axis=-1, keepdims=True) - jnp.square(mu)
    hn = (h - mu) * jax.lax.rsqrt(var + 1e-5) * g_ref[...] + bt_ref[...]
    hr = jnp.maximum(hn, 0.0)
    out_ref[...] = jnp.dot(hr, w2t_ref[...],
                           preferred_element_type=jnp.float32) + b2_ref[...]


@jax.jit
def _tc_mlp(acc, mask, row0, w1t, b1, gamma, beta, w2t_pad, b2_pad):
    bm = 512
    grid = (BATCH // bm,)
    return pl.pallas_call(
        _mlp_body,
        grid=grid,
        in_specs=[
            pl.BlockSpec((bm, EMBED_DIM), lambda i: (i, 0)),
            pl.BlockSpec((bm, SEQ), lambda i: (i, 0)),
            pl.BlockSpec((1, EMBED_DIM), lambda i: (0, 0)),
            pl.BlockSpec((EMBED_DIM, HIDDEN), lambda i: (0, 0)),
            pl.BlockSpec((1, HIDDEN), lambda i: (0, 0)),
            pl.BlockSpec((1, HIDDEN), lambda i: (0, 0)),
            pl.BlockSpec((1, HIDDEN), lambda i: (0, 0)),
            pl.BlockSpec((HIDDEN, HIDDEN), lambda i: (0, 0)),
            pl.BlockSpec((1, HIDDEN), lambda i: (0, 0)),
        ],
        out_specs=pl.BlockSpec((bm, HIDDEN), lambda i: (i, 0)),
        out_shape=jax.ShapeDtypeStruct((BATCH, HIDDEN), jnp.float32),
    )(acc, mask, row0, w1t, b1, gamma, beta, w2t_pad, b2_pad)


def kernel(x, mask, table, W1, b1, gamma, beta, W2, b2):
    x = x.astype(jnp.int32)
    safe_x = jnp.where(mask, 0, x)
    safe_x = jnp.pad(safe_x, ((0, 0), (0, SEQ_PAD - SEQ)))
    acc = _sc_pool(safe_x, table)

    row0 = table[0].reshape(1, EMBED_DIM)
    w1t = W1.T
    w2t_pad = jnp.zeros((HIDDEN, HIDDEN), jnp.float32).at[:, :NUM_CLASSES].set(W2.T)
    b2_pad = jnp.zeros((1, HIDDEN), jnp.float32).at[:, :NUM_CLASSES].set(b2)
    out = _tc_mlp(acc, mask, row0, w1t, b1.reshape(1, -1),
                  gamma.reshape(1, -1), beta.reshape(1, -1), w2t_pad, b2_pad)
    return out[:, :NUM_CLASSES]
